# Initial kernel scaffold; baseline (speedup 1.0000x reference)
import jax
import jax.numpy as jnp
from jax.experimental import pallas as pl


def kernel(expr, src_ctx_tissue, tgt_ctx_tissue, src_ctx_assay, tgt_ctx_assay, W_base, W_enc_tissue, W_dec_tissue, W_heads_tissue, W_enc_assay, W_dec_assay, W_heads_assay):
    raise NotImplementedError("write your pallas kernel here")


# speedup vs baseline: 1.9107x; 1.9107x over previous
import functools

import jax
import jax.numpy as jnp
from jax import lax
from jax.experimental import pallas as pl
from jax.experimental.pallas import tpu as pltpu

B, D, L, H = 2048, 1024, 768, 8
BLK = 256
SCALE = 0.0159

_INTERPRET = False


def _ids_from_ctx(ctx):
    return jnp.argmax(ctx, axis=1)


def _dense_body(x_ref, sct, tct, sca, tca, wb, wet, wdt, wht, wea, wda, wha,
                o_ref):
    x = x_ref[...]
    wbv = wb[...]
    h_base = lax.dot_general(x, wbv, (((1,), (1,)), ((), ())),
                             preferred_element_type=jnp.float32)
    out = lax.dot_general(h_base, wbv, (((1,), (0,)), ((), ())),
                          preferred_element_type=jnp.float32)
    for (sc_ref, tc_ref, we, wd, wh) in ((sct, tct, wet, wdt, wht),
                                         (sca, tca, wea, wda, wha)):
        fdt = we.dtype
        xf = x.astype(fdt)
        shared = lax.dot_general(xf, we[...], (((1,), (1,)), ((), ())),
                                 preferred_element_type=jnp.float32)
        ids_s = _ids_from_ctx(sc_ref[...])
        ids_t = _ids_from_ctx(tc_ref[...])
        sh = shared.astype(fdt)
        h = jnp.zeros((BLK, L), jnp.float32)
        for c in range(H):
            p = lax.dot_general(sh, wh[c], (((1,), (1,)), ((), ())),
                                preferred_element_type=jnp.float32)
            h = h + jnp.where((ids_s == c)[:, None], p, 0.0)
        hh = h.astype(fdt)
        dec = jnp.zeros((BLK, L), jnp.float32)
        for c in range(H):
            p = lax.dot_general(hh, wh[c], (((1,), (1,)), ((), ())),
                                preferred_element_type=jnp.float32)
            dec = dec + jnp.where((ids_t == c)[:, None], p, 0.0)
        out = out + SCALE * lax.dot_general(
            dec.astype(fdt), wd[...], (((1,), (1,)), ((), ())),
            preferred_element_type=jnp.float32)
    o_ref[...] = out


def kernel(expr, src_ctx_tissue, tgt_ctx_tissue, src_ctx_assay, tgt_ctx_assay,
           W_base, W_enc_tissue, W_dec_tissue, W_heads_tissue,
           W_enc_assay, W_dec_assay, W_heads_assay):
    fdt = jnp.bfloat16
    wet = W_enc_tissue.astype(fdt)
    wdt = W_dec_tissue.astype(fdt)
    wht = W_heads_tissue.astype(fdt)
    wea = W_enc_assay.astype(fdt)
    wda = W_dec_assay.astype(fdt)
    wha = W_heads_assay.astype(fdt)

    nblk = B // BLK
    row = lambda i: (i, 0)
    full2 = lambda i: (0, 0)
    full3 = lambda i: (0, 0, 0)
    grid_spec = pl.GridSpec(
        grid=(nblk,),
        in_specs=[
            pl.BlockSpec((BLK, D), row),
            pl.BlockSpec((BLK, H), row),
            pl.BlockSpec((BLK, H), row),
            pl.BlockSpec((BLK, H), row),
            pl.BlockSpec((BLK, H), row),
            pl.BlockSpec((L, D), full2),
            pl.BlockSpec((L, D), full2),
            pl.BlockSpec((D, L), full2),
            pl.BlockSpec((H, L, L), full3),
            pl.BlockSpec((L, D), full2),
            pl.BlockSpec((D, L), full2),
            pl.BlockSpec((H, L, L), full3),
        ],
        out_specs=pl.BlockSpec((BLK, D), row),
    )
    return pl.pallas_call(
        _dense_body,
        grid_spec=grid_spec,
        out_shape=jax.ShapeDtypeStruct((B, D), jnp.float32),
        interpret=_INTERPRET,
    )(expr, src_ctx_tissue, tgt_ctx_tissue, src_ctx_assay, tgt_ctx_assay,
      W_base, wet, wdt, wht, wea, wda, wha)
